# baseline (speedup 1.0000x reference)
import functools

import jax
import jax.numpy as jnp
from jax import lax
from jax.experimental import pallas as pl
from jax.experimental.pallas import tpu as pltpu
from jax.experimental.pallas import tpu_sc as plsc

N = 10000
E = 320000
F = 128
H = 4
C = 16

D = 16
NW = 32
CH = 128
NCH = 2 * (-(-E // (NW * CH * 2)))
EPW = NCH * CH
EPAD = NW * EPW
NGARB = N
ZR = 640
NP = ZR * 16
FW = NP // NW

_mesh = plsc.VectorSubcoreMesh(core_axis_name="c", subcore_axis_name="s")
_sc_params = pltpu.CompilerParams(use_tc_tiling_on_sc=False,
                                  needs_layout_passes=False)


def _newton_rsqrt(d):
    i = lax.bitcast_convert_type(d, jnp.int32)
    i = jnp.int32(0x5F3759DF) - lax.shift_right_arithmetic(i, 1)
    y = lax.bitcast_convert_type(i, jnp.float32)
    for _ in range(4):
        y = y * (1.5 - 0.5 * d * y * y)
    return y


def _zero_fill(zb):
    def zrow(i, c):
        zb[i, :] = jnp.zeros((D,), jnp.float32)
        return c

    lax.fori_loop(0, ZR, zrow, 0)


def _edge_pass(wid, src_hbm, dst_hbm, srcv, dstv, rows0, rows1, ytab_sh, acc,
               sem0, sem1):
    def fire(j, buf, sem):
        pltpu.async_copy(ytab_sh.at[srcv.at[j]], buf, sem)

    def drain(buf, sem):
        pltpu.make_async_copy(ytab_sh.at[srcv.at[0]], buf, sem).wait()

    def scat(j, buf):
        pltpu.sync_copy(buf, acc.at[dstv.at[j]], add=True)

    pltpu.sync_copy(src_hbm.at[wid], srcv)
    pltpu.sync_copy(dst_hbm.at[wid], dstv)
    fire(0, rows0, sem0)
    fire(1, rows1, sem1)

    def pair(j2, c):
        a = 2 * j2
        drain(rows0, sem0)
        scat(a, rows0)
        fire(a + 2, rows0, sem0)
        drain(rows1, sem1)
        scat(a + 1, rows1)
        fire(a + 3, rows1, sem1)
        return c

    lax.fori_loop(0, NCH // 2 - 1, pair, 0)
    drain(rows0, sem0)
    scat(NCH - 2, rows0)
    drain(rows1, sem1)
    scat(NCH - 1, rows1)


@functools.partial(
    pl.kernel,
    out_type=jax.ShapeDtypeStruct((2, NP, D), jnp.float32),
    mesh=_mesh,
    compiler_params=_sc_params,
    scratch_types=[
        pltpu.VMEM((NCH, CH), jnp.int32),
        pltpu.VMEM((CH, D), jnp.float32),
        pltpu.VMEM((ZR, D), jnp.float32),
        pltpu.VMEM_SHARED((NP, D), jnp.float32),
    ],
)
def _sc_degree(dst_hbm, out_hbm, dstv, rows, zb, acc):
    cid = lax.axis_index("c")
    sid = lax.axis_index("s")
    wid = sid * 2 + cid
    sl = pl.ds(sid * ZR, ZR)

    _zero_fill(zb)
    pltpu.sync_copy(zb, acc.at[sl])
    plsc.subcore_barrier()

    pltpu.sync_copy(dst_hbm.at[wid], dstv)

    def orow(i, c):
        rows[i, :] = jnp.ones((D,), jnp.float32)
        return c

    lax.fori_loop(0, CH, orow, 0)

    def step(j, c):
        pltpu.sync_copy(rows, acc.at[dstv.at[j]], add=True)
        return c

    lax.fori_loop(0, NCH, step, 0)

    plsc.subcore_barrier()
    pltpu.sync_copy(acc.at[sl], zb)
    pltpu.sync_copy(zb, out_hbm.at[cid, sl])


@functools.partial(
    pl.kernel,
    out_type=[
        jax.ShapeDtypeStruct((2, NP, D), jnp.float32),
        jax.ShapeDtypeStruct((NP, D), jnp.float32),
        jax.ShapeDtypeStruct((NP, D), jnp.float32),
    ],
    mesh=_mesh,
    compiler_params=_sc_params,
    scratch_types=[
        pltpu.VMEM((NCH, CH), jnp.int32),
        pltpu.VMEM((NCH, CH), jnp.int32),
        pltpu.VMEM((CH, D), jnp.float32),
        pltpu.VMEM((CH, D), jnp.float32),
        pltpu.VMEM((ZR, D), jnp.float32),
        pltpu.VMEM((ZR, D), jnp.float32),
        pltpu.VMEM((ZR, D), jnp.float32),
        pltpu.VMEM((ZR, D), jnp.float32),
        pltpu.VMEM((ZR, D), jnp.float32),
        pltpu.VMEM_SHARED((NP, D), jnp.float32),
        pltpu.VMEM_SHARED((NP, D), jnp.float32),
        pltpu.SemaphoreType.DMA,
        pltpu.SemaphoreType.DMA,
    ],
)
def _sc_layer1(src_hbm, dst_hbm, xw_hbm, degp_hbm, out_hbm, y1_hbm, dis_hbm,
               srcv, dstv, rows0, rows1, zb, xwv, dg0, dg1, y1v, acc, ytab_sh,
               sem0, sem1):
    cid = lax.axis_index("c")
    sid = lax.axis_index("s")
    wid = sid * 2 + cid
    sl = pl.ds(sid * ZR, ZR)

    _zero_fill(zb)
    pltpu.sync_copy(zb, acc.at[sl])
    pltpu.sync_copy(xw_hbm.at[sl], xwv)
    pltpu.sync_copy(degp_hbm.at[0, sl], dg0)
    pltpu.sync_copy(degp_hbm.at[1, sl], dg1)

    def rowfn(r, c):
        d = dg0[r, :] + dg1[r, :] + 1.0
        q = _newton_rsqrt(d)
        dg0[r, :] = q
        y1v[r, :] = q * xwv[r, :]
        return c

    lax.fori_loop(0, ZR, rowfn, 0)
    pltpu.sync_copy(y1v, ytab_sh.at[sl])

    @pl.when(cid == 0)
    def _():
        pltpu.sync_copy(y1v, y1_hbm.at[sl])
        pltpu.sync_copy(dg0, dis_hbm.at[sl])

    plsc.subcore_barrier()
    _edge_pass(wid, src_hbm, dst_hbm, srcv, dstv, rows0, rows1, ytab_sh, acc,
               sem0, sem1)
    plsc.subcore_barrier()
    pltpu.sync_copy(acc.at[sl], zb)
    pltpu.sync_copy(zb, out_hbm.at[cid, sl])


@functools.partial(
    pl.kernel,
    out_type=[
        jax.ShapeDtypeStruct((2, NP, D), jnp.float32),
        jax.ShapeDtypeStruct((NP, D), jnp.float32),
    ],
    mesh=_mesh,
    compiler_params=_sc_params,
    scratch_types=[
        pltpu.VMEM((NCH, CH), jnp.int32),
        pltpu.VMEM((NCH, CH), jnp.int32),
        pltpu.VMEM((CH, D), jnp.float32),
        pltpu.VMEM((CH, D), jnp.float32),
        pltpu.VMEM((ZR, D), jnp.float32),
        pltpu.VMEM((ZR, D), jnp.float32),
        pltpu.VMEM((ZR, D), jnp.float32),
        pltpu.VMEM((ZR, D), jnp.float32),
        pltpu.VMEM((ZR, D), jnp.float32),
        pltpu.VMEM((D, D), jnp.float32),
        pltpu.VMEM((D,), jnp.float32),
        pltpu.VMEM((2 * D,), jnp.float32),
        pltpu.VMEM_SHARED((NP, D), jnp.float32),
        pltpu.VMEM_SHARED((NP, D), jnp.float32),
        pltpu.SemaphoreType.DMA,
        pltpu.SemaphoreType.DMA,
    ],
)
def _sc_layer2(src_hbm, dst_hbm, y1_hbm, dis_hbm, ag1_hbm, w2_hbm, b1_hbm,
               out_hbm, y2_hbm, srcv, dstv, rows0, rows1, zb, y1v, dsv, a0,
               a1, w2v, b1v, hrow, acc, ytab_sh, sem0, sem1):
    cid = lax.axis_index("c")
    sid = lax.axis_index("s")
    wid = sid * 2 + cid
    sl = pl.ds(sid * ZR, ZR)

    _zero_fill(zb)
    pltpu.sync_copy(zb, acc.at[sl])
    pltpu.sync_copy(y1_hbm.at[sl], y1v)
    pltpu.sync_copy(dis_hbm.at[sl], dsv)
    pltpu.sync_copy(ag1_hbm.at[0, sl], a0)
    pltpu.sync_copy(ag1_hbm.at[1, sl], a1)
    pltpu.sync_copy(w2_hbm, w2v)
    pltpu.sync_copy(b1_hbm, b1v)

    def rowfn(r, c):
        pre = dsv[r, :] * (a0[r, :] + a1[r, :] + y1v[r, :]) + b1v[:]
        hrow[pl.ds(8, D)] = jnp.maximum(pre, 0.0)
        o = jnp.zeros((D,), jnp.float32)
        for j in range(H):
            hj = plsc.load_gather(hrow, [jnp.full((D,), 8 + j, jnp.int32)])
            o = o + hj * w2v[j, :]
        y1v[r, :] = dsv[r, :] * o
        return c

    lax.fori_loop(0, ZR, rowfn, 0)
    pltpu.sync_copy(y1v, ytab_sh.at[sl])

    @pl.when(cid == 0)
    def _():
        pltpu.sync_copy(y1v, y2_hbm.at[sl])

    plsc.subcore_barrier()
    _edge_pass(wid, src_hbm, dst_hbm, srcv, dstv, rows0, rows1, ytab_sh, acc,
               sem0, sem1)
    plsc.subcore_barrier()
    pltpu.sync_copy(acc.at[sl], zb)
    pltpu.sync_copy(zb, out_hbm.at[cid, sl])


@functools.partial(
    pl.kernel,
    out_type=[
        jax.ShapeDtypeStruct((NP, D), jnp.float32),
        jax.ShapeDtypeStruct((NP, D), jnp.float32),
    ],
    mesh=_mesh,
    compiler_params=_sc_params,
    scratch_types=[
        pltpu.VMEM((FW, D), jnp.float32),
        pltpu.VMEM((FW, D), jnp.float32),
        pltpu.VMEM((FW, D), jnp.float32),
        pltpu.VMEM((FW, D), jnp.float32),
        pltpu.VMEM((FW, D), jnp.float32),
        pltpu.VMEM((FW, D), jnp.float32),
        pltpu.VMEM((D,), jnp.float32),
    ],
)
def _sc_finalize(y2_hbm, dis_hbm, ag2_hbm, b2_hbm, lg_hbm, pb_hbm, y2v, dsv,
                 a0, a1, lgv, pbv, b2v):
    cid = lax.axis_index("c")
    sid = lax.axis_index("s")
    wid = sid * 2 + cid
    fsl = pl.ds(wid * FW, FW)

    pltpu.sync_copy(y2_hbm.at[fsl], y2v)
    pltpu.sync_copy(dis_hbm.at[fsl], dsv)
    pltpu.sync_copy(ag2_hbm.at[0, fsl], a0)
    pltpu.sync_copy(ag2_hbm.at[1, fsl], a1)
    pltpu.sync_copy(b2_hbm, b2v)

    def rowfn(r, c):
        lg = dsv[r, :] * (a0[r, :] + a1[r, :] + y2v[r, :]) + b2v[:]
        lgv[r, :] = lg
        m = jnp.max(lg)
        e = jnp.exp(lg - m)
        s = jnp.sum(e)
        pbv[r, :] = e / s
        return c

    lax.fori_loop(0, FW, rowfn, 0)
    pltpu.sync_copy(lgv, lg_hbm.at[fsl])
    pltpu.sync_copy(pbv, pb_hbm.at[fsl])


def _k1a_body(x_ref, w1_ref, o_ref):
    o_ref[: N, :] = jnp.dot(x_ref[...], w1_ref[...],
                            preferred_element_type=jnp.float32)
    o_ref[N:, :] = jnp.zeros((NP - N, D), jnp.float32)


_k1a = pl.pallas_call(
    _k1a_body,
    out_shape=jax.ShapeDtypeStruct((NP, D), jnp.float32),
)


def kernel(x, edge_index, W1, b1, W2, b2):
    src = edge_index[0]
    dst = edge_index[1]
    pad = EPAD - E
    srcp = jnp.concatenate([src, jnp.zeros((pad,), jnp.int32)])
    srcp = srcp.reshape(NW, NCH, CH)
    dstp = jnp.concatenate([dst, jnp.full((pad,), NGARB, jnp.int32)])
    dstp = dstp.reshape(NW, NCH, CH)

    W1p = jnp.pad(W1, ((0, 0), (0, D - H)))
    b1p = jnp.pad(b1, (0, D - H))
    W2p = jnp.pad(W2, ((0, D - H), (0, 0)))

    degp = _sc_degree(dstp)
    xw = _k1a(x, W1p)
    agg1p, y1tab, distab = _sc_layer1(srcp, dstp, xw, degp)
    agg2p, y2tab = _sc_layer2(srcp, dstp, y1tab, distab, agg1p, W2p, b1p)
    lg, pb = _sc_finalize(y2tab, distab, agg2p, b2)
    return lg[: N], pb[: N]
